# baseline (speedup 1.0000x reference)
import functools

import jax
import jax.numpy as jnp
import numpy as np
from jax import lax
from jax.experimental import pallas as pl
from jax.experimental.pallas import tpu as pltpu
from jax.experimental.pallas import tpu_sc as plsc

IMG_H, IMG_W = 608, 800
STRIDE = 16
FEAT_H, FEAT_W = IMG_H // STRIDE, IMG_W // STRIDE
N_CLASS = 21
PRE_NMS = 6000
POST_NMS = 300
NMS_THRESH = 0.7
POOL = 7
FC_DIM = 1024

N_ANCHOR = FEAT_H * FEAT_W * 9
NMS_R, NMS_C = 136, 128
NMS_PAD = NMS_R * NMS_C
KEEP_R, KEEP_C = 8, 128
M_PAD = 304
K_TOTAL = 512 * POOL * POOL
NK = 7
K_CHUNK = K_TOTAL // NK


def _conv(x, W, b, stride=1, pad=1):
    out = lax.conv_general_dilated(x, W, (stride, stride), [(pad, pad), (pad, pad)],
                                   dimension_numbers=('NCHW', 'OIHW', 'NCHW'))
    return out + b[None, :, None, None]


def _anchors():
    base = 16.0
    anc = []
    for r in [0.5, 1.0, 2.0]:
        for s in [8.0, 16.0, 32.0]:
            h = base * s * np.sqrt(r)
            w = base * s * np.sqrt(1.0 / r)
            anc.append([-h / 2.0, -w / 2.0, h / 2.0, w / 2.0])
    base_anc = np.asarray(anc, dtype=np.float32)
    sy = (np.arange(FEAT_H) * STRIDE).astype(np.float32)
    sx = (np.arange(FEAT_W) * STRIDE).astype(np.float32)
    yy, xx = np.meshgrid(sy, sx, indexing='ij')
    shifts = np.stack([yy, xx, yy, xx], -1).reshape(-1, 1, 4)
    return jnp.asarray((shifts + base_anc[None]).reshape(-1, 4))


def _loc2bbox(src, loc):
    h = src[:, 2] - src[:, 0]
    w = src[:, 3] - src[:, 1]
    cy = src[:, 0] + 0.5 * h
    cx = src[:, 1] + 0.5 * w
    dy, dx = loc[:, 0], loc[:, 1]
    dh = jnp.clip(loc[:, 2], -4.0, 4.0)
    dw = jnp.clip(loc[:, 3], -4.0, 4.0)
    ncy = dy * h + cy
    ncx = dx * w + cx
    nh = jnp.exp(dh) * h
    nw = jnp.exp(dw) * w
    return jnp.stack([ncy - 0.5 * nh, ncx - 0.5 * nw, ncy + 0.5 * nh, ncx + 0.5 * nw], 1)


def _nms_kernel(y1_ref, x1_ref, y2_ref, x2_ref, sc_ref, keep_ref):
    y1 = y1_ref[...]
    x1 = x1_ref[...]
    y2 = y2_ref[...]
    x2 = x2_ref[...]
    fg = sc_ref[...]
    areas = (y2 - y1) * (x2 - x1)
    idx2d = (lax.broadcasted_iota(jnp.int32, (NMS_R, NMS_C), 0) * NMS_C
             + lax.broadcasted_iota(jnp.int32, (NMS_R, NMS_C), 1))
    kiota = (lax.broadcasted_iota(jnp.int32, (KEEP_R, KEEP_C), 0) * KEEP_C
             + lax.broadcasted_iota(jnp.int32, (KEEP_R, KEEP_C), 1))
    neg_inf = jnp.float32(-jnp.inf)
    real = idx2d < N_ANCHOR

    keys = jnp.where(real & (fg != neg_inf),
                     lax.bitcast_convert_type(fg, jnp.int32),
                     jnp.where(real, -1, -2))

    def bs_body(_, lohi):
        lo, hi = lohi
        mid = lo + (hi - lo) // 2
        cnt = jnp.sum(jnp.where(keys >= mid, 1, 0))
        big = cnt >= PRE_NMS
        return jnp.where(big, mid, lo), jnp.where(big, hi, mid)

    lo, _ = lax.fori_loop(0, 32, bs_body,
                          (jnp.int32(-2), jnp.int32(0x3f800002)))
    scores0 = jnp.where(keys >= lo, fg, neg_inf)

    m0 = jnp.max(scores0)
    i0 = jnp.min(jnp.where(scores0 == m0, idx2d, NMS_PAD))

    def body(t, carry):
        scores, keep = carry
        m = jnp.max(scores)
        i = jnp.where(m == neg_inf, i0,
                      jnp.min(jnp.where(scores == m, idx2d, NMS_PAD)))
        sel = idx2d == i
        by1 = jnp.sum(jnp.where(sel, y1, 0.0))
        bx1 = jnp.sum(jnp.where(sel, x1, 0.0))
        by2 = jnp.sum(jnp.where(sel, y2, 0.0))
        bx2 = jnp.sum(jnp.where(sel, x2, 0.0))
        barea = jnp.sum(jnp.where(sel, areas, 0.0))
        ty = jnp.maximum(by1, y1)
        tx = jnp.maximum(bx1, x1)
        by = jnp.minimum(by2, y2)
        bx = jnp.minimum(bx2, x2)
        inter = jnp.maximum(by - ty, 0.0) * jnp.maximum(bx - tx, 0.0)
        iou = inter / (barea + areas - inter + 1e-9)
        scores = jnp.where(iou > NMS_THRESH, neg_inf, scores)
        keep = jnp.where(kiota == t, i, keep)
        return scores, keep

    _, keep = lax.fori_loop(
        0, POST_NMS, body,
        (scores0, jnp.zeros((KEEP_R, KEEP_C), jnp.int32)))
    keep_ref[...] = keep


def _run_nms(props, fg):
    pad = NMS_PAD - N_ANCHOR
    boxes = jnp.pad(props, ((0, pad), (0, 0)))
    sc = jnp.pad(fg, (0, pad), constant_values=-jnp.inf)
    y1 = boxes[:, 0].reshape(NMS_R, NMS_C)
    x1 = boxes[:, 1].reshape(NMS_R, NMS_C)
    y2 = boxes[:, 2].reshape(NMS_R, NMS_C)
    x2 = boxes[:, 3].reshape(NMS_R, NMS_C)
    sc2 = sc.reshape(NMS_R, NMS_C)
    keep2d = pl.pallas_call(
        _nms_kernel,
        out_shape=jax.ShapeDtypeStruct((KEEP_R, KEEP_C), jnp.int32),
    )(y1, x1, y2, x2, sc2)
    keep = keep2d.reshape(-1)[:POST_NMS]
    return boxes[keep]


NPTS = POST_NMS * POOL * POOL
SC_NW = 32
SC_CHUNK = 32
SC_NCHUNK = 15
SC_PER_W = SC_CHUNK * SC_NCHUNK
NPTS_PAD = SC_PER_W * SC_NW
FEAT_C = 512


@functools.partial(
    pl.kernel,
    mesh=plsc.VectorSubcoreMesh(core_axis_name="c", subcore_axis_name="s"),
    out_type=jax.ShapeDtypeStruct((NPTS_PAD, FEAT_C), jnp.float32),
    scratch_types=[
        pltpu.VMEM((SC_CHUNK,), jnp.int32),
        pltpu.VMEM((SC_CHUNK,), jnp.int32),
        pltpu.VMEM((SC_CHUNK,), jnp.int32),
        pltpu.VMEM((SC_CHUNK,), jnp.int32),
        pltpu.VMEM((SC_CHUNK, 16), jnp.float32),
        pltpu.VMEM((SC_CHUNK, 16), jnp.float32),
        pltpu.VMEM((SC_CHUNK, 16), jnp.float32),
        pltpu.VMEM((SC_CHUNK, 16), jnp.float32),
        pltpu.VMEM((SC_CHUNK, FEAT_C), jnp.float32),
        pltpu.VMEM((SC_CHUNK, FEAT_C), jnp.float32),
        pltpu.VMEM((SC_CHUNK, FEAT_C), jnp.float32),
        pltpu.VMEM((SC_CHUNK, FEAT_C), jnp.float32),
        pltpu.VMEM((SC_CHUNK, FEAT_C), jnp.float32),
        pltpu.SemaphoreType.DMA,
    ],
)
def _sc_roialign(feat_hbm, i0_hbm, i1_hbm, i2_hbm, i3_hbm,
                 w0_hbm, w1_hbm, w2_hbm, w3_hbm, out_hbm,
                 i0_v, i1_v, i2_v, i3_v, w0_v, w1_v, w2_v, w3_v,
                 r0_v, r1_v, r2_v, r3_v, out_v, sem):
    wid = lax.axis_index("s") * 2 + lax.axis_index("c")
    base = wid * SC_PER_W
    for c in range(SC_NCHUNK):
        off = base + c * SC_CHUNK
        pltpu.sync_copy(i0_hbm.at[pl.ds(off, SC_CHUNK)], i0_v)
        pltpu.sync_copy(i1_hbm.at[pl.ds(off, SC_CHUNK)], i1_v)
        pltpu.sync_copy(i2_hbm.at[pl.ds(off, SC_CHUNK)], i2_v)
        pltpu.sync_copy(i3_hbm.at[pl.ds(off, SC_CHUNK)], i3_v)
        pltpu.sync_copy(w0_hbm.at[pl.ds(off, SC_CHUNK), :], w0_v)
        pltpu.sync_copy(w1_hbm.at[pl.ds(off, SC_CHUNK), :], w1_v)
        pltpu.sync_copy(w2_hbm.at[pl.ds(off, SC_CHUNK), :], w2_v)
        pltpu.sync_copy(w3_hbm.at[pl.ds(off, SC_CHUNK), :], w3_v)
        d0 = pltpu.async_copy(feat_hbm.at[i0_v], r0_v, sem)
        d1 = pltpu.async_copy(feat_hbm.at[i1_v], r1_v, sem)
        d2 = pltpu.async_copy(feat_hbm.at[i2_v], r2_v, sem)
        d3 = pltpu.async_copy(feat_hbm.at[i3_v], r3_v, sem)
        d0.wait()
        d1.wait()
        d2.wait()
        d3.wait()

        def pt_body(p, _):
            w0 = w0_v[p, :]
            w1 = w1_v[p, :]
            w2 = w2_v[p, :]
            w3 = w3_v[p, :]
            for s in range(FEAT_C // 16):
                sl = pl.ds(s * 16, 16)
                out_v[p, sl] = (r0_v[p, sl] * w0 + r1_v[p, sl] * w1
                                + r2_v[p, sl] * w2 + r3_v[p, sl] * w3)
            return 0

        lax.fori_loop(0, SC_CHUNK, pt_body, 0)
        pltpu.sync_copy(out_v, out_hbm.at[pl.ds(off, SC_CHUNK)])


def _roi_align_sc(feat_t, rois):
    H, W = FEAT_H, FEAT_W
    R = rois.shape[0]
    y1 = rois[:, 0] / STRIDE
    x1 = rois[:, 1] / STRIDE
    bh = (rois[:, 2] - rois[:, 0]) / STRIDE / POOL
    bw = (rois[:, 3] - rois[:, 1]) / STRIDE / POOL
    g = jnp.arange(POOL, dtype=jnp.float32) + 0.5
    py = jnp.clip(y1[:, None] + bh[:, None] * g[None], 0.0, H - 1.0)
    px = jnp.clip(x1[:, None] + bw[:, None] * g[None], 0.0, W - 1.0)
    y0 = jnp.floor(py)
    x0 = jnp.floor(px)
    wy = (py - y0)[:, :, None]
    wx = (px - x0)[:, None, :]
    y0i = y0.astype(jnp.int32)
    x0i = x0.astype(jnp.int32)
    y1i = jnp.minimum(y0i + 1, H - 1)
    x1i = jnp.minimum(x0i + 1, W - 1)
    Y0 = jnp.broadcast_to(y0i[:, :, None], (R, POOL, POOL)).reshape(-1)
    Y1 = jnp.broadcast_to(y1i[:, :, None], (R, POOL, POOL)).reshape(-1)
    X0 = jnp.broadcast_to(x0i[:, None, :], (R, POOL, POOL)).reshape(-1)
    X1 = jnp.broadcast_to(x1i[:, None, :], (R, POOL, POOL)).reshape(-1)
    w00 = ((1 - wy) * (1 - wx)).reshape(-1)
    w01 = ((1 - wy) * wx).reshape(-1)
    w10 = (wy * (1 - wx)).reshape(-1)
    w11 = (wy * wx).reshape(-1)
    pad = NPTS_PAD - NPTS
    val = _sc_roialign(
        feat_t,
        jnp.pad(Y0 * W + X0, (0, pad)), jnp.pad(Y0 * W + X1, (0, pad)),
        jnp.pad(Y1 * W + X0, (0, pad)), jnp.pad(Y1 * W + X1, (0, pad)),
        jnp.broadcast_to(jnp.pad(w00, (0, pad))[:, None], (NPTS_PAD, 16)),
        jnp.broadcast_to(jnp.pad(w01, (0, pad))[:, None], (NPTS_PAD, 16)),
        jnp.broadcast_to(jnp.pad(w10, (0, pad))[:, None], (NPTS_PAD, 16)),
        jnp.broadcast_to(jnp.pad(w11, (0, pad))[:, None], (NPTS_PAD, 16)))
    val = val[:NPTS]
    return jnp.transpose(val.reshape(R, POOL * POOL, FEAT_C), (0, 2, 1)).reshape(R, -1)


def _roi_align_rows(feat_t, rois):
    H, W = FEAT_H, FEAT_W
    C = feat_t.shape[1]
    R = rois.shape[0]
    y1 = rois[:, 0] / STRIDE
    x1 = rois[:, 1] / STRIDE
    bh = (rois[:, 2] - rois[:, 0]) / STRIDE / POOL
    bw = (rois[:, 3] - rois[:, 1]) / STRIDE / POOL
    g = jnp.arange(POOL, dtype=jnp.float32) + 0.5
    py = jnp.clip(y1[:, None] + bh[:, None] * g[None], 0.0, H - 1.0)
    px = jnp.clip(x1[:, None] + bw[:, None] * g[None], 0.0, W - 1.0)
    y0 = jnp.floor(py)
    x0 = jnp.floor(px)
    wy = (py - y0)[:, :, None]
    wx = (px - x0)[:, None, :]
    y0i = y0.astype(jnp.int32)
    x0i = x0.astype(jnp.int32)
    y1i = jnp.minimum(y0i + 1, H - 1)
    x1i = jnp.minimum(x0i + 1, W - 1)
    Y0 = jnp.broadcast_to(y0i[:, :, None], (R, POOL, POOL)).reshape(-1)
    Y1 = jnp.broadcast_to(y1i[:, :, None], (R, POOL, POOL)).reshape(-1)
    X0 = jnp.broadcast_to(x0i[:, None, :], (R, POOL, POOL)).reshape(-1)
    X1 = jnp.broadcast_to(x1i[:, None, :], (R, POOL, POOL)).reshape(-1)
    w00 = ((1 - wy) * (1 - wx)).reshape(-1, 1)
    w01 = ((1 - wy) * wx).reshape(-1, 1)
    w10 = (wy * (1 - wx)).reshape(-1, 1)
    w11 = (wy * wx).reshape(-1, 1)
    val = (feat_t[Y0 * W + X0] * w00 + feat_t[Y0 * W + X1] * w01
           + feat_t[Y1 * W + X0] * w10 + feat_t[Y1 * W + X1] * w11)
    return jnp.transpose(val.reshape(R, POOL * POOL, C), (0, 2, 1)).reshape(R, -1)


def _head_kernel(pooled_ref, wf1_ref, bf1_ref, wf2_ref, bf2_ref,
                 wcl_ref, bcl_ref, wsc_ref, bsc_ref,
                 cl_ref, sc_ref, acc_ref):
    k = pl.program_id(0)

    @pl.when(k == 0)
    def _init():
        acc_ref[...] = jnp.zeros_like(acc_ref)

    acc_ref[...] += jnp.dot(pooled_ref[...], wf1_ref[...],
                            preferred_element_type=jnp.float32)

    @pl.when(k == NK - 1)
    def _finish():
        f = jnp.maximum(acc_ref[...] + bf1_ref[...], 0.0)
        f = jnp.maximum(jnp.dot(f, wf2_ref[...],
                                preferred_element_type=jnp.float32) + bf2_ref[...], 0.0)
        cl_ref[...] = jnp.dot(f, wcl_ref[...],
                              preferred_element_type=jnp.float32) + bcl_ref[...]
        sc_ref[...] = jnp.dot(f, wsc_ref[...],
                              preferred_element_type=jnp.float32) + bsc_ref[...]


def _run_head(pooled, Wf1, bf1, Wf2, bf2, Wcl, bcl, Wsc, bsc):
    pooled = jnp.pad(pooled, ((0, M_PAD - POST_NMS), (0, 0)))
    cl, sc = pl.pallas_call(
        _head_kernel,
        grid=(NK,),
        in_specs=[
            pl.BlockSpec((M_PAD, K_CHUNK), lambda k: (0, k)),
            pl.BlockSpec((K_CHUNK, FC_DIM), lambda k: (k, 0)),
            pl.BlockSpec((1, FC_DIM), lambda k: (0, 0)),
            pl.BlockSpec((FC_DIM, FC_DIM), lambda k: (0, 0)),
            pl.BlockSpec((1, FC_DIM), lambda k: (0, 0)),
            pl.BlockSpec((FC_DIM, N_CLASS * 4), lambda k: (0, 0)),
            pl.BlockSpec((1, N_CLASS * 4), lambda k: (0, 0)),
            pl.BlockSpec((FC_DIM, N_CLASS), lambda k: (0, 0)),
            pl.BlockSpec((1, N_CLASS), lambda k: (0, 0)),
        ],
        out_specs=[
            pl.BlockSpec((M_PAD, N_CLASS * 4), lambda k: (0, 0)),
            pl.BlockSpec((M_PAD, N_CLASS), lambda k: (0, 0)),
        ],
        out_shape=[
            jax.ShapeDtypeStruct((M_PAD, N_CLASS * 4), jnp.float32),
            jax.ShapeDtypeStruct((M_PAD, N_CLASS), jnp.float32),
        ],
        scratch_shapes=[pltpu.VMEM((M_PAD, FC_DIM), jnp.float32)],
    )(pooled, Wf1, bf1.reshape(1, -1), Wf2, bf2.reshape(1, -1),
      Wcl, bcl.reshape(1, -1), Wsc, bsc.reshape(1, -1))
    return cl[:POST_NMS], sc[:POST_NMS]


def kernel(x, W1, b1, W2, b2, W3, b3, W4, b4, Wr, br, Wl, bl, Ws, bs,
           Wf1, bf1, Wf2, bf2, Wcl, bcl, Wsc, bsc):
    h = jax.nn.relu(_conv(x, W1, b1, 2))
    h = jax.nn.relu(_conv(h, W2, b2, 2))
    h = jax.nn.relu(_conv(h, W3, b3, 2))
    h = jax.nn.relu(_conv(h, W4, b4, 2))
    rh = jax.nn.relu(_conv(h, Wr, br, 1))
    locs = jnp.transpose(_conv(rh, Wl, bl, 1, 0), (0, 2, 3, 1)).reshape(-1, 4)
    sc2 = jnp.transpose(_conv(rh, Ws, bs, 1, 0), (0, 2, 3, 1)).reshape(-1, 2)
    fg = jax.nn.softmax(sc2, axis=1)[:, 1]
    props = _loc2bbox(_anchors(), locs)
    props = jnp.stack([jnp.clip(props[:, 0], 0.0, float(IMG_H)),
                       jnp.clip(props[:, 1], 0.0, float(IMG_W)),
                       jnp.clip(props[:, 2], 0.0, float(IMG_H)),
                       jnp.clip(props[:, 3], 0.0, float(IMG_W))], 1)
    valid = ((props[:, 2] - props[:, 0]) >= 16.0) & ((props[:, 3] - props[:, 1]) >= 16.0)
    fg = jnp.where(valid, fg, -jnp.inf)
    rois = _run_nms(lax.stop_gradient(props), lax.stop_gradient(fg))
    feat_t = jnp.transpose(h[0].reshape(512, FEAT_H * FEAT_W), (1, 0))
    pooled = _roi_align_sc(feat_t, rois)
    return _run_head(pooled, Wf1, bf1, Wf2, bf2, Wcl, bcl, Wsc, bsc)


# speedup vs baseline: 1.0437x; 1.0437x over previous
import functools

import jax
import jax.numpy as jnp
import numpy as np
from jax import lax
from jax.experimental import pallas as pl
from jax.experimental.pallas import tpu as pltpu
from jax.experimental.pallas import tpu_sc as plsc

IMG_H, IMG_W = 608, 800
STRIDE = 16
FEAT_H, FEAT_W = IMG_H // STRIDE, IMG_W // STRIDE
N_CLASS = 21
PRE_NMS = 6000
POST_NMS = 300
NMS_THRESH = 0.7
POOL = 7
FC_DIM = 1024

N_ANCHOR = FEAT_H * FEAT_W * 9
NMS_R, NMS_C = 136, 128
NMS_PAD = NMS_R * NMS_C
KEEP_R, KEEP_C = 8, 128
M_PAD = 304
K_TOTAL = 512 * POOL * POOL
NK = 7
K_CHUNK = K_TOTAL // NK


def _conv(x, W, b, stride=1, pad=1):
    out = lax.conv_general_dilated(x, W, (stride, stride), [(pad, pad), (pad, pad)],
                                   dimension_numbers=('NCHW', 'OIHW', 'NCHW'))
    return out + b[None, :, None, None]


def _anchors():
    base = 16.0
    anc = []
    for r in [0.5, 1.0, 2.0]:
        for s in [8.0, 16.0, 32.0]:
            h = base * s * np.sqrt(r)
            w = base * s * np.sqrt(1.0 / r)
            anc.append([-h / 2.0, -w / 2.0, h / 2.0, w / 2.0])
    base_anc = np.asarray(anc, dtype=np.float32)
    sy = (np.arange(FEAT_H) * STRIDE).astype(np.float32)
    sx = (np.arange(FEAT_W) * STRIDE).astype(np.float32)
    yy, xx = np.meshgrid(sy, sx, indexing='ij')
    shifts = np.stack([yy, xx, yy, xx], -1).reshape(-1, 1, 4)
    return jnp.asarray((shifts + base_anc[None]).reshape(-1, 4))


def _loc2bbox(src, loc):
    h = src[:, 2] - src[:, 0]
    w = src[:, 3] - src[:, 1]
    cy = src[:, 0] + 0.5 * h
    cx = src[:, 1] + 0.5 * w
    dy, dx = loc[:, 0], loc[:, 1]
    dh = jnp.clip(loc[:, 2], -4.0, 4.0)
    dw = jnp.clip(loc[:, 3], -4.0, 4.0)
    ncy = dy * h + cy
    ncx = dx * w + cx
    nh = jnp.exp(dh) * h
    nw = jnp.exp(dw) * w
    return jnp.stack([ncy - 0.5 * nh, ncx - 0.5 * nw, ncy + 0.5 * nh, ncx + 0.5 * nw], 1)


def _nms_kernel(y1_ref, x1_ref, y2_ref, x2_ref, sc_ref, keep_ref):
    y1 = y1_ref[...]
    x1 = x1_ref[...]
    y2 = y2_ref[...]
    x2 = x2_ref[...]
    fg = sc_ref[...]
    areas = (y2 - y1) * (x2 - x1)
    idx2d = (lax.broadcasted_iota(jnp.int32, (NMS_R, NMS_C), 0) * NMS_C
             + lax.broadcasted_iota(jnp.int32, (NMS_R, NMS_C), 1))
    kiota = (lax.broadcasted_iota(jnp.int32, (KEEP_R, KEEP_C), 0) * KEEP_C
             + lax.broadcasted_iota(jnp.int32, (KEEP_R, KEEP_C), 1))
    neg_inf = jnp.float32(-jnp.inf)
    real = idx2d < N_ANCHOR

    keys = jnp.where(real & (fg != neg_inf),
                     lax.bitcast_convert_type(fg, jnp.int32),
                     jnp.where(real, -1, -2))

    def bs_body(_, lohi):
        lo, hi = lohi
        mid = lo + (hi - lo) // 2
        cnt = jnp.sum(jnp.where(keys >= mid, 1, 0))
        big = cnt >= PRE_NMS
        return jnp.where(big, mid, lo), jnp.where(big, hi, mid)

    lo, _ = lax.fori_loop(0, 32, bs_body,
                          (jnp.int32(-2), jnp.int32(0x3f800002)))
    scores0 = jnp.where(keys >= lo, fg, neg_inf)

    m0 = jnp.max(scores0)
    i0 = jnp.min(jnp.where(scores0 == m0, idx2d, NMS_PAD))

    def body(t, carry):
        scores, keep = carry
        m = jnp.max(scores)
        i = jnp.where(m == neg_inf, i0,
                      jnp.min(jnp.where(scores == m, idx2d, NMS_PAD)))
        sel = idx2d == i
        by1 = jnp.sum(jnp.where(sel, y1, 0.0))
        bx1 = jnp.sum(jnp.where(sel, x1, 0.0))
        by2 = jnp.sum(jnp.where(sel, y2, 0.0))
        bx2 = jnp.sum(jnp.where(sel, x2, 0.0))
        barea = jnp.sum(jnp.where(sel, areas, 0.0))
        ty = jnp.maximum(by1, y1)
        tx = jnp.maximum(bx1, x1)
        by = jnp.minimum(by2, y2)
        bx = jnp.minimum(bx2, x2)
        inter = jnp.maximum(by - ty, 0.0) * jnp.maximum(bx - tx, 0.0)
        iou = inter / (barea + areas - inter + 1e-9)
        scores = jnp.where(iou > NMS_THRESH, neg_inf, scores)
        keep = jnp.where(kiota == t, i, keep)
        return scores, keep

    _, keep = lax.fori_loop(
        0, POST_NMS, body,
        (scores0, jnp.zeros((KEEP_R, KEEP_C), jnp.int32)))
    keep_ref[...] = keep


def _run_nms(props, fg):
    pad = NMS_PAD - N_ANCHOR
    boxes = jnp.pad(props, ((0, pad), (0, 0)))
    sc = jnp.pad(fg, (0, pad), constant_values=-jnp.inf)
    y1 = boxes[:, 0].reshape(NMS_R, NMS_C)
    x1 = boxes[:, 1].reshape(NMS_R, NMS_C)
    y2 = boxes[:, 2].reshape(NMS_R, NMS_C)
    x2 = boxes[:, 3].reshape(NMS_R, NMS_C)
    sc2 = sc.reshape(NMS_R, NMS_C)
    keep2d = pl.pallas_call(
        _nms_kernel,
        out_shape=jax.ShapeDtypeStruct((KEEP_R, KEEP_C), jnp.int32),
    )(y1, x1, y2, x2, sc2)
    keep = keep2d.reshape(-1)[:POST_NMS]
    return boxes[keep]


NPTS = POST_NMS * POOL * POOL
SC_NW = 32
SC_CHUNK = 16
SC_NCHUNK = 30
SC_PER_W = SC_CHUNK * SC_NCHUNK
NPTS_PAD = SC_PER_W * SC_NW
FEAT_C = 512


@functools.partial(
    pl.kernel,
    mesh=plsc.VectorSubcoreMesh(core_axis_name="c", subcore_axis_name="s"),
    out_type=jax.ShapeDtypeStruct((NPTS_PAD, FEAT_C), jnp.float32),
    scratch_types=[
        pltpu.VMEM((SC_PER_W,), jnp.int32),
        pltpu.VMEM((SC_PER_W,), jnp.int32),
        pltpu.VMEM((SC_PER_W,), jnp.int32),
        pltpu.VMEM((SC_PER_W,), jnp.int32),
        pltpu.VMEM((SC_PER_W * 16,), jnp.float32),
        pltpu.VMEM((SC_PER_W * 16,), jnp.float32),
        pltpu.VMEM((SC_PER_W * 16,), jnp.float32),
        pltpu.VMEM((SC_PER_W * 16,), jnp.float32),
        pltpu.VMEM((2, 4, SC_CHUNK, FEAT_C), jnp.float32),
        pltpu.VMEM((SC_CHUNK, FEAT_C), jnp.float32),
        pltpu.SemaphoreType.DMA,
        pltpu.SemaphoreType.DMA,
    ],
)
def _sc_roialign(feat_hbm, i0_hbm, i1_hbm, i2_hbm, i3_hbm,
                 w0_hbm, w1_hbm, w2_hbm, w3_hbm, out_hbm,
                 i0_v, i1_v, i2_v, i3_v, w0_v, w1_v, w2_v, w3_v,
                 rbuf, out_v, semA, semB):
    wid = lax.axis_index("s") * 2 + lax.axis_index("c")
    base = wid * SC_PER_W
    pltpu.sync_copy(i0_hbm.at[pl.ds(base, SC_PER_W)], i0_v)
    pltpu.sync_copy(i1_hbm.at[pl.ds(base, SC_PER_W)], i1_v)
    pltpu.sync_copy(i2_hbm.at[pl.ds(base, SC_PER_W)], i2_v)
    pltpu.sync_copy(i3_hbm.at[pl.ds(base, SC_PER_W)], i3_v)
    wbase = base * 16
    pltpu.sync_copy(w0_hbm.at[pl.ds(wbase, SC_PER_W * 16)], w0_v)
    pltpu.sync_copy(w1_hbm.at[pl.ds(wbase, SC_PER_W * 16)], w1_v)
    pltpu.sync_copy(w2_hbm.at[pl.ds(wbase, SC_PER_W * 16)], w2_v)
    pltpu.sync_copy(w3_hbm.at[pl.ds(wbase, SC_PER_W * 16)], w3_v)
    sems = (semA, semB)
    irefs = (i0_v, i1_v, i2_v, i3_v)
    wrefs = (w0_v, w1_v, w2_v, w3_v)

    def fire(c, slot):
        lo = pl.ds(c * SC_CHUNK, SC_CHUNK)
        for j in range(4):
            pltpu.async_copy(feat_hbm.at[irefs[j].at[lo]],
                             rbuf.at[slot, j], sems[slot])

    def wait_slot(c, slot):
        lo = pl.ds(c * SC_CHUNK, SC_CHUNK)
        for j in range(4):
            pltpu.make_async_copy(feat_hbm.at[irefs[j].at[lo]],
                                  rbuf.at[slot, j], sems[slot]).wait()

    def combine_write(c, slot):
        def pt_body(p, _):
            q = (c * SC_CHUNK + p) * 16
            w0 = w0_v[pl.ds(q, 16)]
            w1 = w1_v[pl.ds(q, 16)]
            w2 = w2_v[pl.ds(q, 16)]
            w3 = w3_v[pl.ds(q, 16)]
            for s in range(FEAT_C // 16):
                sl = pl.ds(s * 16, 16)
                out_v[p, sl] = (rbuf[slot, 0, p, sl] * w0
                                + rbuf[slot, 1, p, sl] * w1
                                + rbuf[slot, 2, p, sl] * w2
                                + rbuf[slot, 3, p, sl] * w3)
            return 0

        lax.fori_loop(0, SC_CHUNK, pt_body, 0)
        pltpu.sync_copy(out_v, out_hbm.at[pl.ds(base + c * SC_CHUNK, SC_CHUNK)])

    fire(0, 0)
    fire(1, 1)

    def gbody(g, _):
        c0 = 2 * g
        wait_slot(c0, 0)
        combine_write(c0, 0)
        fire(c0 + 2, 0)
        c1 = 2 * g + 1
        wait_slot(c1, 1)
        combine_write(c1, 1)
        fire(c1 + 2, 1)
        return 0

    lax.fori_loop(0, SC_NCHUNK // 2 - 1, gbody, 0)
    wait_slot(SC_NCHUNK - 2, 0)
    combine_write(SC_NCHUNK - 2, 0)
    wait_slot(SC_NCHUNK - 1, 1)
    combine_write(SC_NCHUNK - 1, 1)


def _roi_align_sc(feat_t, rois):
    H, W = FEAT_H, FEAT_W
    R = rois.shape[0]
    y1 = rois[:, 0] / STRIDE
    x1 = rois[:, 1] / STRIDE
    bh = (rois[:, 2] - rois[:, 0]) / STRIDE / POOL
    bw = (rois[:, 3] - rois[:, 1]) / STRIDE / POOL
    g = jnp.arange(POOL, dtype=jnp.float32) + 0.5
    py = jnp.clip(y1[:, None] + bh[:, None] * g[None], 0.0, H - 1.0)
    px = jnp.clip(x1[:, None] + bw[:, None] * g[None], 0.0, W - 1.0)
    y0 = jnp.floor(py)
    x0 = jnp.floor(px)
    wy = (py - y0)[:, :, None]
    wx = (px - x0)[:, None, :]
    y0i = y0.astype(jnp.int32)
    x0i = x0.astype(jnp.int32)
    y1i = jnp.minimum(y0i + 1, H - 1)
    x1i = jnp.minimum(x0i + 1, W - 1)
    Y0 = jnp.broadcast_to(y0i[:, :, None], (R, POOL, POOL)).reshape(-1)
    Y1 = jnp.broadcast_to(y1i[:, :, None], (R, POOL, POOL)).reshape(-1)
    X0 = jnp.broadcast_to(x0i[:, None, :], (R, POOL, POOL)).reshape(-1)
    X1 = jnp.broadcast_to(x1i[:, None, :], (R, POOL, POOL)).reshape(-1)
    w00 = ((1 - wy) * (1 - wx)).reshape(-1)
    w01 = ((1 - wy) * wx).reshape(-1)
    w10 = (wy * (1 - wx)).reshape(-1)
    w11 = (wy * wx).reshape(-1)
    pad = NPTS_PAD - NPTS
    val = _sc_roialign(
        feat_t,
        jnp.pad(Y0 * W + X0, (0, pad)), jnp.pad(Y0 * W + X1, (0, pad)),
        jnp.pad(Y1 * W + X0, (0, pad)), jnp.pad(Y1 * W + X1, (0, pad)),
        jnp.broadcast_to(jnp.pad(w00, (0, pad))[:, None], (NPTS_PAD, 16)).reshape(-1),
        jnp.broadcast_to(jnp.pad(w01, (0, pad))[:, None], (NPTS_PAD, 16)).reshape(-1),
        jnp.broadcast_to(jnp.pad(w10, (0, pad))[:, None], (NPTS_PAD, 16)).reshape(-1),
        jnp.broadcast_to(jnp.pad(w11, (0, pad))[:, None], (NPTS_PAD, 16)).reshape(-1))
    val = val[:NPTS]
    return jnp.transpose(val.reshape(R, POOL * POOL, FEAT_C), (0, 2, 1)).reshape(R, -1)


def _roi_align_rows(feat_t, rois):
    H, W = FEAT_H, FEAT_W
    C = feat_t.shape[1]
    R = rois.shape[0]
    y1 = rois[:, 0] / STRIDE
    x1 = rois[:, 1] / STRIDE
    bh = (rois[:, 2] - rois[:, 0]) / STRIDE / POOL
    bw = (rois[:, 3] - rois[:, 1]) / STRIDE / POOL
    g = jnp.arange(POOL, dtype=jnp.float32) + 0.5
    py = jnp.clip(y1[:, None] + bh[:, None] * g[None], 0.0, H - 1.0)
    px = jnp.clip(x1[:, None] + bw[:, None] * g[None], 0.0, W - 1.0)
    y0 = jnp.floor(py)
    x0 = jnp.floor(px)
    wy = (py - y0)[:, :, None]
    wx = (px - x0)[:, None, :]
    y0i = y0.astype(jnp.int32)
    x0i = x0.astype(jnp.int32)
    y1i = jnp.minimum(y0i + 1, H - 1)
    x1i = jnp.minimum(x0i + 1, W - 1)
    Y0 = jnp.broadcast_to(y0i[:, :, None], (R, POOL, POOL)).reshape(-1)
    Y1 = jnp.broadcast_to(y1i[:, :, None], (R, POOL, POOL)).reshape(-1)
    X0 = jnp.broadcast_to(x0i[:, None, :], (R, POOL, POOL)).reshape(-1)
    X1 = jnp.broadcast_to(x1i[:, None, :], (R, POOL, POOL)).reshape(-1)
    w00 = ((1 - wy) * (1 - wx)).reshape(-1, 1)
    w01 = ((1 - wy) * wx).reshape(-1, 1)
    w10 = (wy * (1 - wx)).reshape(-1, 1)
    w11 = (wy * wx).reshape(-1, 1)
    val = (feat_t[Y0 * W + X0] * w00 + feat_t[Y0 * W + X1] * w01
           + feat_t[Y1 * W + X0] * w10 + feat_t[Y1 * W + X1] * w11)
    return jnp.transpose(val.reshape(R, POOL * POOL, C), (0, 2, 1)).reshape(R, -1)


def _head_kernel(pooled_ref, wf1_ref, bf1_ref, wf2_ref, bf2_ref,
                 wcl_ref, bcl_ref, wsc_ref, bsc_ref,
                 cl_ref, sc_ref, acc_ref):
    k = pl.program_id(0)

    @pl.when(k == 0)
    def _init():
        acc_ref[...] = jnp.zeros_like(acc_ref)

    acc_ref[...] += jnp.dot(pooled_ref[...], wf1_ref[...],
                            preferred_element_type=jnp.float32)

    @pl.when(k == NK - 1)
    def _finish():
        f = jnp.maximum(acc_ref[...] + bf1_ref[...], 0.0)
        f = jnp.maximum(jnp.dot(f, wf2_ref[...],
                                preferred_element_type=jnp.float32) + bf2_ref[...], 0.0)
        cl_ref[...] = jnp.dot(f, wcl_ref[...],
                              preferred_element_type=jnp.float32) + bcl_ref[...]
        sc_ref[...] = jnp.dot(f, wsc_ref[...],
                              preferred_element_type=jnp.float32) + bsc_ref[...]


def _run_head(pooled, Wf1, bf1, Wf2, bf2, Wcl, bcl, Wsc, bsc):
    pooled = jnp.pad(pooled, ((0, M_PAD - POST_NMS), (0, 0)))
    cl, sc = pl.pallas_call(
        _head_kernel,
        grid=(NK,),
        in_specs=[
            pl.BlockSpec((M_PAD, K_CHUNK), lambda k: (0, k)),
            pl.BlockSpec((K_CHUNK, FC_DIM), lambda k: (k, 0)),
            pl.BlockSpec((1, FC_DIM), lambda k: (0, 0)),
            pl.BlockSpec((FC_DIM, FC_DIM), lambda k: (0, 0)),
            pl.BlockSpec((1, FC_DIM), lambda k: (0, 0)),
            pl.BlockSpec((FC_DIM, N_CLASS * 4), lambda k: (0, 0)),
            pl.BlockSpec((1, N_CLASS * 4), lambda k: (0, 0)),
            pl.BlockSpec((FC_DIM, N_CLASS), lambda k: (0, 0)),
            pl.BlockSpec((1, N_CLASS), lambda k: (0, 0)),
        ],
        out_specs=[
            pl.BlockSpec((M_PAD, N_CLASS * 4), lambda k: (0, 0)),
            pl.BlockSpec((M_PAD, N_CLASS), lambda k: (0, 0)),
        ],
        out_shape=[
            jax.ShapeDtypeStruct((M_PAD, N_CLASS * 4), jnp.float32),
            jax.ShapeDtypeStruct((M_PAD, N_CLASS), jnp.float32),
        ],
        scratch_shapes=[pltpu.VMEM((M_PAD, FC_DIM), jnp.float32)],
    )(pooled, Wf1, bf1.reshape(1, -1), Wf2, bf2.reshape(1, -1),
      Wcl, bcl.reshape(1, -1), Wsc, bsc.reshape(1, -1))
    return cl[:POST_NMS], sc[:POST_NMS]


def kernel(x, W1, b1, W2, b2, W3, b3, W4, b4, Wr, br, Wl, bl, Ws, bs,
           Wf1, bf1, Wf2, bf2, Wcl, bcl, Wsc, bsc):
    h = jax.nn.relu(_conv(x, W1, b1, 2))
    h = jax.nn.relu(_conv(h, W2, b2, 2))
    h = jax.nn.relu(_conv(h, W3, b3, 2))
    h = jax.nn.relu(_conv(h, W4, b4, 2))
    rh = jax.nn.relu(_conv(h, Wr, br, 1))
    locs = jnp.transpose(_conv(rh, Wl, bl, 1, 0), (0, 2, 3, 1)).reshape(-1, 4)
    sc2 = jnp.transpose(_conv(rh, Ws, bs, 1, 0), (0, 2, 3, 1)).reshape(-1, 2)
    fg = jax.nn.softmax(sc2, axis=1)[:, 1]
    props = _loc2bbox(_anchors(), locs)
    props = jnp.stack([jnp.clip(props[:, 0], 0.0, float(IMG_H)),
                       jnp.clip(props[:, 1], 0.0, float(IMG_W)),
                       jnp.clip(props[:, 2], 0.0, float(IMG_H)),
                       jnp.clip(props[:, 3], 0.0, float(IMG_W))], 1)
    valid = ((props[:, 2] - props[:, 0]) >= 16.0) & ((props[:, 3] - props[:, 1]) >= 16.0)
    fg = jnp.where(valid, fg, -jnp.inf)
    rois = _run_nms(lax.stop_gradient(props), lax.stop_gradient(fg))
    feat_t = jnp.transpose(h[0].reshape(512, FEAT_H * FEAT_W), (1, 0))
    pooled = _roi_align_sc(feat_t, rois)
    return _run_head(pooled, Wf1, bf1, Wf2, bf2, Wcl, bcl, Wsc, bsc)
